# baseline (speedup 1.0000x reference)
import functools

import jax
import jax.numpy as jnp
from jax import lax
from jax.experimental import pallas as pl
from jax.experimental.pallas import tpu as pltpu
from jax.experimental.pallas import tpu_sc as plsc

_NT_DIM = 64
_EMBED_DIM = 128
_OUT_DIM = 2 * _EMBED_DIM + _NT_DIM

_NC = 2
_NS = 16
_NW = _NC * _NS
_CH = 64
_BLK = 2 * _CH
_LANES = 16
_TRH = _OUT_DIM // 2


def _feature_kernel(L, B):
    N = L * B
    rows_per_w = N // _NW
    nch = rows_per_w // _CH
    nblk = nch // 2
    blocks_per_l = B // _BLK

    mesh = plsc.VectorSubcoreMesh(core_axis_name="c", subcore_axis_name="s")

    def buf_set():
        return [
            pltpu.VMEM((4, _CH), jnp.int32),
            pltpu.VMEM((_CH, _EMBED_DIM), jnp.float32),
            pltpu.VMEM((_CH, _EMBED_DIM), jnp.float32),
            pltpu.VMEM((_CH, _EMBED_DIM), jnp.float32),
            pltpu.VMEM((_CH, _EMBED_DIM), jnp.float32),
            pltpu.SemaphoreType.DMA,
            pltpu.SemaphoreType.DMA,
        ]

    @functools.partial(
        pl.kernel,
        out_type=(jax.ShapeDtypeStruct((L, _OUT_DIM, B), jnp.float32),
                  jax.ShapeDtypeStruct((1024, _EMBED_DIM), jnp.float32)),
        mesh=mesh,
        compiler_params=pltpu.CompilerParams(needs_layout_passes=False),
        scratch_types=buf_set() + buf_set() + [
            pltpu.VMEM((_TRH, _BLK), jnp.float32),
            pltpu.VMEM((_TRH, _BLK), jnp.float32),
            pltpu.VMEM((32, _NT_DIM), jnp.float32),
            pltpu.SemaphoreType.DMA,
            pltpu.SemaphoreType.DMA,
        ],
    )
    def body(idx_hbm, rule_hbm, token_hbm, ntab_hbm, out_hbm, ntab2_hbm,
             *scratch):
        bufs = (scratch[:7], scratch[7:14])
        tr_a, tr_b, stage_v, sp, so = scratch[14:19]
        wid = lax.axis_index("s") * _NC + lax.axis_index("c")
        ch0 = wid * nch
        blk0 = wid * nblk
        lane = lax.iota(jnp.int32, _LANES)
        c0 = lane * 0

        def idx_copy(c, S):
            return pltpu.make_async_copy(idx_hbm.at[ch0 + c], S[0], S[5])

        def g_copies(S):
            idx, sg = S[0], S[6]
            return (
                pltpu.make_async_copy(rule_hbm.at[idx.at[0]], S[1], sg),
                pltpu.make_async_copy(token_hbm.at[idx.at[1]], S[2], sg),
                pltpu.make_async_copy(ntab2_hbm.at[idx.at[2]], S[3], sg),
                pltpu.make_async_copy(rule_hbm.at[idx.at[3]], S[4], sg),
            )

        def blk_copies(k):
            gk = blk0 + k
            l = gk // blocks_per_l
            bcol = (gk % blocks_per_l) * _BLK
            return (
                pltpu.make_async_copy(
                    tr_a, out_hbm.at[l, pl.ds(0, _TRH), pl.ds(bcol, _BLK)],
                    so),
                pltpu.make_async_copy(
                    tr_b, out_hbm.at[l, pl.ds(_TRH, _TRH), pl.ds(bcol, _BLK)],
                    so),
            )

        def transpose_wave(S, w):
            rule_v, tok_v, node_v, par_v = S[1], S[2], S[3], S[4]
            woff = w * _CH

            def sum_col(d, rc):
                cd = c0 + d
                for j in range(_CH // _LANES):
                    rj = lane + j * _LANES
                    v = (plsc.load_gather(rule_v, [rj, cd])
                         + plsc.load_gather(tok_v, [rj, cd]))
                    tr_a[d, pl.ds(woff + j * _LANES, _LANES)] = v
                return rc

            def node_lo_col(d, rc):
                cd = c0 + d
                for j in range(_CH // _LANES):
                    rj = lane + j * _LANES
                    v = plsc.load_gather(node_v, [rj, cd])
                    tr_a[_EMBED_DIM + d, pl.ds(woff + j * _LANES, _LANES)] = v
                return rc

            def node_hi_col(d, rc):
                cd = c0 + (d + 32)
                for j in range(_CH // _LANES):
                    rj = lane + j * _LANES
                    v = plsc.load_gather(node_v, [rj, cd])
                    tr_b[d, pl.ds(woff + j * _LANES, _LANES)] = v
                return rc

            def par_col(d, rc):
                cd = c0 + d
                for j in range(_CH // _LANES):
                    rj = lane + j * _LANES
                    v = plsc.load_gather(par_v, [rj, cd])
                    tr_b[32 + d, pl.ds(woff + j * _LANES, _LANES)] = v
                return rc

            lax.fori_loop(0, _EMBED_DIM, sum_col, 0)
            lax.fori_loop(0, 32, node_lo_col, 0)
            lax.fori_loop(0, 32, node_hi_col, 0)
            lax.fori_loop(0, _EMBED_DIM, par_col, 0)

        idx_copy(0, bufs[0]).start()
        idx_copy(1, bufs[1]).start()

        tid = lax.axis_index("s")
        r0 = 8 * jnp.minimum(8 * tid, 117)

        def widen_rows(r, rc):
            for j in range(_NT_DIM // _LANES):
                sl = pl.ds(j * _LANES, _LANES)
                tr_a[r, sl] = stage_v[r, sl]
            return rc

        for k in range(2):
            pltpu.make_async_copy(
                ntab_hbm.at[pl.ds(r0 + 32 * k, 32)], stage_v, sp).start()
            pltpu.make_async_copy(
                ntab_hbm.at[pl.ds(0, 32)], stage_v, sp).wait()
            lax.fori_loop(0, 32, widen_rows, 0)
            pltpu.make_async_copy(
                tr_a.at[pl.ds(0, 32), pl.ds(0, _EMBED_DIM)],
                ntab2_hbm.at[pl.ds(r0 + 32 * k, 32)], sp).start()
            pltpu.make_async_copy(
                tr_a.at[pl.ds(0, 32), pl.ds(0, _EMBED_DIM)],
                ntab2_hbm.at[pl.ds(0, 32)], sp).wait()
        plsc.subcore_barrier()

        idx_copy(0, bufs[0]).wait()
        for d in g_copies(bufs[0]):
            d.start()

        def step(i, carry):
            for b in (0, 1):
                S, T = bufs[b], bufs[1 - b]
                c = i * 2 + b
                if b == 0:
                    @pl.when(i >= 1)
                    def _():
                        for d in blk_copies(i - 1):
                            d.wait()
                if b == 0:
                    idx_copy(c + 1, T).wait()
                    for d in g_copies(T):
                        d.start()
                else:
                    @pl.when(i < nblk - 1)
                    def _():
                        idx_copy(c + 1, T).wait()
                        for d in g_copies(T):
                            d.start()
                for d in g_copies(S):
                    d.wait()

                @pl.when(i < nblk - 1)
                def _():
                    idx_copy(c + 2, S).start()

                transpose_wave(S, b)
                if b == 1:
                    for d in blk_copies(i):
                        d.start()
            return carry

        lax.fori_loop(0, nblk, step, 0)
        for d in blk_copies(nblk - 1):
            d.wait()

    return body


def kernel(actions, previous_actions, rule_embed, token_embed, node_type_embed):
    L, B, _ = actions.shape
    N = L * B
    a = actions.reshape(N, 3)
    p = previous_actions.reshape(N, 3)

    idx_all = jnp.stack([p[:, 0], p[:, 1], a[:, 0], a[:, 1]], axis=0)
    idx_all = idx_all.reshape(4, N // _CH, _CH).transpose(1, 0, 2)

    out3, _ = _feature_kernel(L, B)(
        idx_all, rule_embed, token_embed, node_type_embed)
    return out3.transpose(0, 2, 1), actions[:, :, 2]


# speedup vs baseline: 2.8206x; 2.8206x over previous
import functools

import jax
import jax.numpy as jnp
from jax import lax
from jax.experimental import pallas as pl
from jax.experimental.pallas import tpu as pltpu
from jax.experimental.pallas import tpu_sc as plsc

_NT_DIM = 64
_EMBED_DIM = 128
_OUT_DIM = 2 * _EMBED_DIM + _NT_DIM

_NC = 2
_NS = 16
_NW = _NC * _NS
_CH = 64
_LANES = 16
_RGRP = 8


def _feature_kernel(N):
    rows_per_w = N // _NW
    nch = rows_per_w // _CH
    mesh = plsc.VectorSubcoreMesh(core_axis_name="c", subcore_axis_name="s")

    def buf_set():
        return [
            pltpu.VMEM((4, _CH), jnp.int32),
            pltpu.VMEM((_CH, _OUT_DIM), jnp.float32),
            pltpu.VMEM((_CH, _EMBED_DIM), jnp.float32),
            pltpu.VMEM((_CH, _EMBED_DIM), jnp.float32),
            pltpu.SemaphoreType.DMA,
            pltpu.SemaphoreType.DMA,
            pltpu.SemaphoreType.DMA,
        ]

    @functools.partial(
        pl.kernel,
        out_type=(jax.ShapeDtypeStruct((N, _OUT_DIM), jnp.float32),
                  jax.ShapeDtypeStruct((1024, _EMBED_DIM), jnp.float32)),
        mesh=mesh,
        compiler_params=pltpu.CompilerParams(needs_layout_passes=False),
        scratch_types=buf_set() + buf_set() + [
            pltpu.VMEM((32, _NT_DIM), jnp.float32),
            pltpu.SemaphoreType.DMA,
        ],
    )
    def body(idx_hbm, rule_hbm, token_hbm, ntab_hbm, out_hbm, ntab2_hbm,
             *scratch):
        bufs = (scratch[:7], scratch[7:14])
        stage_v, sp = scratch[14], scratch[15]
        wid = lax.axis_index("s") * _NC + lax.axis_index("c")
        ch0 = wid * nch

        def idx_copy(c, S):
            return pltpu.make_async_copy(idx_hbm.at[ch0 + c], S[0], S[4])

        def g_copies(S):
            idx, out, tok, par, sg = S[0], S[1], S[2], S[3], S[5]
            return (
                pltpu.make_async_copy(
                    rule_hbm.at[idx.at[0]],
                    out.at[:, pl.ds(0, _EMBED_DIM)], sg),
                pltpu.make_async_copy(token_hbm.at[idx.at[1]], tok, sg),
                pltpu.make_async_copy(
                    ntab2_hbm.at[idx.at[2]],
                    out.at[:, pl.ds(_EMBED_DIM, _EMBED_DIM)], sg),
                pltpu.make_async_copy(rule_hbm.at[idx.at[3]], par, sg),
            )

        def out_copy(c, S):
            return pltpu.make_async_copy(
                S[1], out_hbm.at[pl.ds((ch0 + c) * _CH, _CH)], S[6])

        def repack(S):
            out, tok, par = S[1], S[2], S[3]

            def rows(g, rc):
                for rr in range(_RGRP):
                    r = g * _RGRP + rr
                    for j in range(_EMBED_DIM // _LANES):
                        sl = pl.ds(j * _LANES, _LANES)
                        out[r, sl] = out[r, sl] + tok[r, sl]
                    for j in range(_EMBED_DIM // _LANES):
                        dst = pl.ds(_EMBED_DIM + _NT_DIM + j * _LANES, _LANES)
                        out[r, dst] = par[r, pl.ds(j * _LANES, _LANES)]
                return rc

            lax.fori_loop(0, _CH // _RGRP, rows, 0)

        idx_copy(0, bufs[0]).start()
        idx_copy(1, bufs[1]).start()

        tid = lax.axis_index("s")
        r0 = 8 * jnp.minimum(8 * tid, 117)
        stage_out = bufs[0][1]

        def widen_rows(r, rc):
            for j in range(_NT_DIM // _LANES):
                sl = pl.ds(j * _LANES, _LANES)
                stage_out[r, sl] = stage_v[r, sl]
            return rc

        for k in range(2):
            pltpu.make_async_copy(
                ntab_hbm.at[pl.ds(r0 + 32 * k, 32)], stage_v, sp).start()
            pltpu.make_async_copy(
                ntab_hbm.at[pl.ds(0, 32)], stage_v, sp).wait()
            lax.fori_loop(0, 32, widen_rows, 0)
            pltpu.make_async_copy(
                stage_out.at[pl.ds(0, 32), pl.ds(0, _EMBED_DIM)],
                ntab2_hbm.at[pl.ds(r0 + 32 * k, 32)], sp).start()
            pltpu.make_async_copy(
                stage_out.at[pl.ds(0, 32), pl.ds(0, _EMBED_DIM)],
                ntab2_hbm.at[pl.ds(0, 32)], sp).wait()
        plsc.subcore_barrier()

        idx_copy(0, bufs[0]).wait()
        for d in g_copies(bufs[0]):
            d.start()

        def step(i, carry):
            for b in (0, 1):
                S, T = bufs[b], bufs[1 - b]
                c = i * 2 + b
                if b == 0:
                    @pl.when(i >= 1)
                    def _():
                        out_copy(c - 1, T).wait()
                else:
                    out_copy(c - 1, T).wait()
                if b == 0:
                    idx_copy(c + 1, T).wait()
                    for d in g_copies(T):
                        d.start()
                else:
                    @pl.when(i < nch // 2 - 1)
                    def _():
                        idx_copy(c + 1, T).wait()
                        for d in g_copies(T):
                            d.start()
                for d in g_copies(S):
                    d.wait()

                @pl.when(i < nch // 2 - 1)
                def _():
                    idx_copy(c + 2, S).start()

                repack(S)
                out_copy(c, S).start()
            return carry

        lax.fori_loop(0, nch // 2, step, 0)
        out_copy(nch - 1, bufs[1]).wait()

    return body


def kernel(actions, previous_actions, rule_embed, token_embed, node_type_embed):
    L, B, _ = actions.shape
    N = L * B
    a = actions.reshape(N, 3)
    p = previous_actions.reshape(N, 3)

    idx_all = jnp.stack([p[:, 0], p[:, 1], a[:, 0], a[:, 1]], axis=0)
    idx_all = idx_all.reshape(4, N // _CH, _CH).transpose(1, 0, 2)

    feature, _ = _feature_kernel(N)(
        idx_all, rule_embed, token_embed, node_type_embed)
    return feature.reshape(L, B, _OUT_DIM), actions[:, :, 2]
